# baseline (speedup 1.0000x reference)
import functools

import jax
import jax.numpy as jnp
from jax.experimental import pallas as pl
from jax.experimental.pallas import tpu as pltpu


def _dwsep_body(x_ref, halo_ref, wdw_ref, wpw_ref, beff_ref, o_ref, *, k, tl):
    pad = k // 2
    xv = x_ref[0]
    wdw = wdw_ref[...]

    if pad > 0:
        lt = pl.program_id(1)
        halo = halo_ref[lt, 0]
        xe = jnp.concatenate([halo[:, :pad], xv, halo[:, pad:]], axis=1)
    else:
        xe = xv

    acc = xe[:, 0:tl] * wdw[:, 0:1]
    for t in range(1, k):
        acc = acc + xe[:, t:t + tl] * wdw[:, t:t + 1]

    y = jnp.dot(wpw_ref[...], acc.astype(jnp.bfloat16),
                preferred_element_type=jnp.float32) + beff_ref[...]
    o_ref[0] = y.astype(o_ref.dtype)


def _pick_tile(length, cap=2048):
    if length <= cap:
        return length
    t = cap
    while t >= 128:
        if length % t == 0:
            return t
        t -= 128
    return length


def kernel(x, w_dw_pt, b_dw, w_pw_pt, b_pw):
    n, cin, length = x.shape
    cout, _, _ = w_pw_pt.shape
    k = w_dw_pt.shape[2]
    pad = k // 2

    orig_length = length
    if length > 128 and length % 128 != 0:
        new_len = ((length + 127) // 128) * 128
        x = jnp.pad(x, ((0, 0), (0, 0), (0, new_len - length)))
        length = new_len

    tile_l = _pick_tile(length)
    num_tiles = length // tile_l

    w_dw = w_dw_pt[:, 0, :].astype(jnp.float32)
    w_pw = w_pw_pt[:, :, 0]
    b_eff = (b_pw.astype(jnp.float32)
             + w_pw.astype(jnp.float32) @ b_dw.astype(jnp.float32))
    b_eff = b_eff.reshape(cout, 1)
    w_pw_bf = w_pw.astype(jnp.bfloat16)

    hw = max(2 * pad, 2)
    if pad > 0:
        x_t = x.reshape(n, cin, num_tiles, tile_l)
        z = jnp.zeros((n, cin, 1, pad), x.dtype)
        left = jnp.concatenate([z, x_t[:, :, :-1, tile_l - pad:]], axis=2)
        right = jnp.concatenate([x_t[:, :, 1:, :pad], z], axis=2)
        halo = jnp.concatenate([left, right], axis=3)
        halo = jnp.transpose(halo, (2, 0, 1, 3))
    else:
        halo = jnp.zeros((num_tiles, n, cin, hw), x.dtype)

    body = functools.partial(_dwsep_body, k=k, tl=tile_l)

    flops = 2 * n * length * cin * (k + cout)
    bytes_accessed = int((x.size + n * cout * length + halo.size
                          + w_dw.size) * 4 + w_pw.size * 2 + b_eff.size * 4)

    out = pl.pallas_call(
        body,
        out_shape=jax.ShapeDtypeStruct((n, cout, length), x.dtype),
        grid=(n, num_tiles),
        in_specs=[
            pl.BlockSpec((1, cin, tile_l), lambda bi, lt: (bi, 0, lt)),
            pl.BlockSpec((num_tiles, 1, cin, hw), lambda bi, lt: (0, bi, 0, 0)),
            pl.BlockSpec((cin, k), lambda bi, lt: (0, 0)),
            pl.BlockSpec((cout, cin), lambda bi, lt: (0, 0)),
            pl.BlockSpec((cout, 1), lambda bi, lt: (0, 0)),
        ],
        out_specs=pl.BlockSpec((1, cout, tile_l), lambda bi, lt: (bi, 0, lt)),
        compiler_params=pltpu.CompilerParams(
            dimension_semantics=("parallel", "parallel"),
            vmem_limit_bytes=48 * 1024 * 1024),
        cost_estimate=pl.CostEstimate(
            flops=int(flops), transcendentals=0,
            bytes_accessed=bytes_accessed),
    )(x, halo, w_dw, w_pw_bf, b_eff)

    if length != orig_length:
        out = out[:, :, :orig_length]
    return out


# speedup vs baseline: 1.4716x; 1.4716x over previous
import functools

import jax
import jax.numpy as jnp
from jax.experimental import pallas as pl
from jax.experimental.pallas import tpu as pltpu


def _dwsep_body(x_ref, xl_ref, xr_ref, wdw_ref, wpw_ref, beff_ref, o_ref,
                *, k, tl, num_tiles):
    pad = k // 2
    lt = pl.program_id(1)
    xv = x_ref[0]
    wdw = wdw_ref[...]

    if pad > 0:
        lcols = xl_ref[0, :, 128 - pad:]
        rcols = xr_ref[0, :, :pad]
        lcols = jnp.where(lt > 0, lcols, 0.0)
        rcols = jnp.where(lt < num_tiles - 1, rcols, 0.0)
        xe = jnp.concatenate([lcols, xv, rcols], axis=1)
    else:
        xe = xv

    acc = xe[:, 0:tl] * wdw[:, 0:1]
    for t in range(1, k):
        acc = acc + xe[:, t:t + tl] * wdw[:, t:t + 1]

    y = jnp.dot(wpw_ref[...], acc.astype(jnp.bfloat16),
                preferred_element_type=jnp.float32) + beff_ref[...]
    o_ref[0] = y.astype(o_ref.dtype)


def _pick_tile(length, cap=2048):
    if length <= cap:
        return length
    t = cap
    while t >= 128:
        if length % t == 0:
            return t
        t -= 128
    return length


def kernel(x, w_dw_pt, b_dw, w_pw_pt, b_pw):
    n, cin, length = x.shape
    cout, _, _ = w_pw_pt.shape
    k = w_dw_pt.shape[2]
    pad = k // 2
    assert pad < 128

    orig_length = length
    if length % 128 != 0:
        new_len = ((length + 127) // 128) * 128
        x = jnp.pad(x, ((0, 0), (0, 0), (0, new_len - length)))
        length = new_len

    tile_l = _pick_tile(length)
    num_tiles = length // tile_l
    sub = tile_l // 128

    w_dw = w_dw_pt[:, 0, :].astype(jnp.float32)
    w_pw = w_pw_pt[:, :, 0]
    b_eff = (b_pw.astype(jnp.float32)
             + w_pw.astype(jnp.float32) @ b_dw.astype(jnp.float32))
    b_eff = b_eff.reshape(cout, 1)
    w_pw_bf = w_pw.astype(jnp.bfloat16)

    body = functools.partial(_dwsep_body, k=k, tl=tile_l, num_tiles=num_tiles)

    nblk = length // 128

    def lmap(bi, lt):
        return (bi, 0, jnp.maximum(lt * sub - 1, 0))

    def rmap(bi, lt):
        return (bi, 0, jnp.minimum((lt + 1) * sub, nblk - 1))

    flops = 2 * n * length * cin * (k + cout)
    bytes_accessed = int((x.size + n * cout * length
                          + n * num_tiles * cin * 256
                          + w_dw.size) * 4 + w_pw.size * 2 + b_eff.size * 4)

    out = pl.pallas_call(
        body,
        out_shape=jax.ShapeDtypeStruct((n, cout, length), x.dtype),
        grid=(n, num_tiles),
        in_specs=[
            pl.BlockSpec((1, cin, tile_l), lambda bi, lt: (bi, 0, lt)),
            pl.BlockSpec((1, cin, 128), lmap),
            pl.BlockSpec((1, cin, 128), rmap),
            pl.BlockSpec((cin, k), lambda bi, lt: (0, 0)),
            pl.BlockSpec((cout, cin), lambda bi, lt: (0, 0)),
            pl.BlockSpec((cout, 1), lambda bi, lt: (0, 0)),
        ],
        out_specs=pl.BlockSpec((1, cout, tile_l), lambda bi, lt: (bi, 0, lt)),
        compiler_params=pltpu.CompilerParams(
            dimension_semantics=("parallel", "parallel"),
            vmem_limit_bytes=48 * 1024 * 1024),
        cost_estimate=pl.CostEstimate(
            flops=int(flops), transcendentals=0,
            bytes_accessed=bytes_accessed),
    )(x, x, x, w_dw, w_pw_bf, b_eff)

    if length != orig_length:
        out = out[:, :, :orig_length]
    return out


# speedup vs baseline: 2.3550x; 1.6003x over previous
import functools

import jax
import jax.numpy as jnp
from jax.experimental import pallas as pl
from jax.experimental.pallas import tpu as pltpu


def _dwsep_body(x_ref, xl_ref, xr_ref, wdw_ref, wpw_ref, beff_ref, o_ref,
                *, k, tl, num_tiles):
    pad = k // 2
    lt = pl.program_id(1)
    xv = x_ref[0].astype(jnp.bfloat16)
    wdw = wdw_ref[...].astype(jnp.bfloat16)

    if pad > 0:
        lcols = xl_ref[0, :, 128 - pad:].astype(jnp.bfloat16)
        rcols = xr_ref[0, :, :pad].astype(jnp.bfloat16)
        zero = jnp.bfloat16(0)
        lcols = jnp.where(lt > 0, lcols, zero)
        rcols = jnp.where(lt < num_tiles - 1, rcols, zero)
        xe = jnp.concatenate([lcols, xv, rcols], axis=1)
    else:
        xe = xv

    acc = xe[:, 0:tl] * wdw[:, 0:1]
    for t in range(1, k):
        acc = acc + xe[:, t:t + tl] * wdw[:, t:t + 1]

    y = jnp.dot(wpw_ref[...], acc,
                preferred_element_type=jnp.float32) + beff_ref[...]
    o_ref[0] = y.astype(o_ref.dtype)


def _pick_tile(length, cap=2048):
    if length <= cap:
        return length
    t = cap
    while t >= 128:
        if length % t == 0:
            return t
        t -= 128
    return length


def kernel(x, w_dw_pt, b_dw, w_pw_pt, b_pw):
    n, cin, length = x.shape
    cout, _, _ = w_pw_pt.shape
    k = w_dw_pt.shape[2]
    pad = k // 2
    assert pad < 128

    orig_length = length
    if length % 128 != 0:
        new_len = ((length + 127) // 128) * 128
        x = jnp.pad(x, ((0, 0), (0, 0), (0, new_len - length)))
        length = new_len

    tile_l = _pick_tile(length)
    num_tiles = length // tile_l
    sub = tile_l // 128

    w_dw = w_dw_pt[:, 0, :].astype(jnp.float32)
    w_pw = w_pw_pt[:, :, 0]
    b_eff = (b_pw.astype(jnp.float32)
             + w_pw.astype(jnp.float32) @ b_dw.astype(jnp.float32))
    b_eff = b_eff.reshape(cout, 1)
    w_pw_bf = w_pw.astype(jnp.bfloat16)

    body = functools.partial(_dwsep_body, k=k, tl=tile_l, num_tiles=num_tiles)

    nblk = length // 128

    def lmap(bi, lt):
        return (bi, 0, jnp.maximum(lt * sub - 1, 0))

    def rmap(bi, lt):
        return (bi, 0, jnp.minimum((lt + 1) * sub, nblk - 1))

    flops = 2 * n * length * cin * (k + cout)
    bytes_accessed = int((x.size + n * cout * length
                          + n * num_tiles * cin * 256
                          + w_dw.size) * 4 + w_pw.size * 2 + b_eff.size * 4)

    out = pl.pallas_call(
        body,
        out_shape=jax.ShapeDtypeStruct((n, cout, length), x.dtype),
        grid=(n, num_tiles),
        in_specs=[
            pl.BlockSpec((1, cin, tile_l), lambda bi, lt: (bi, 0, lt)),
            pl.BlockSpec((1, cin, 128), lmap),
            pl.BlockSpec((1, cin, 128), rmap),
            pl.BlockSpec((cin, k), lambda bi, lt: (0, 0)),
            pl.BlockSpec((cout, cin), lambda bi, lt: (0, 0)),
            pl.BlockSpec((cout, 1), lambda bi, lt: (0, 0)),
        ],
        out_specs=pl.BlockSpec((1, cout, tile_l), lambda bi, lt: (bi, 0, lt)),
        compiler_params=pltpu.CompilerParams(
            dimension_semantics=("parallel", "parallel"),
            vmem_limit_bytes=48 * 1024 * 1024),
        cost_estimate=pl.CostEstimate(
            flops=int(flops), transcendentals=0,
            bytes_accessed=bytes_accessed),
    )(x, x, x, w_dw, w_pw_bf, b_eff)

    if length != orig_length:
        out = out[:, :, :orig_length]
    return out
